# baseline (speedup 1.0000x reference)
import jax
import jax.numpy as jnp
from jax import lax
from jax.experimental import pallas as pl
from jax.experimental.pallas import tpu as pltpu
from jax.experimental.pallas import tpu_sc as plsc

_N_ROWS = 128
_N_COLS = 128
_N_ACT = 1024
_BATCH = 16384

_NC = 2
_NS = 16
_L = 16
_NW = _NC * _NS
_BPW = _BATCH // _NW
_K = 16
_NG = _BPW // _K
_NROWCHAIN = 4
_STEPS = _N_ACT // _L
_UNROLL = 8
_NBUF = 4


def _sc_body(obs_hbm, tab_hbm, out_hbm, obs_v, idx_v, buf0, buf1, buf2, buf3,
             cval_v, cidx_v, res_v, sem0, sem1, sem2, sem3):
    wid = lax.axis_index("s") * _NC + lax.axis_index("c")
    base = wid * _BPW

    pltpu.sync_copy(obs_hbm.at[pl.ds(base * 2, _BPW * 2)], obs_v)

    iota = lax.iota(jnp.int32, _L)

    bufs = (buf0, buf1, buf2, buf3)
    sems = (sem0, sem1, sem2, sem3)

    def dma(g, p):
        return pltpu.make_async_copy(tab_hbm.at[idx_v.at[g]], bufs[p],
                                     sems[p])

    def flat_ids(g):
        rsel = (g * _L + iota) * 2
        r = plsc.load_gather(obs_v, [rsel])
        c = plsc.load_gather(obs_v, [rsel + 1])
        idx_v[g, :] = r * _N_COLS + c

    for g in range(_NBUF):
        flat_ids(g)
    for p in range(_NBUF):
        dma(p, p).start()
    for g in range(_NBUF, _NG):
        flat_ids(g)

    neg_inf = jnp.full((_L,), -jnp.inf, jnp.float32)
    zeros = jnp.zeros((_L,), jnp.int32)

    def compute_group(g, p):
        buf = bufs[p]
        dma(g, p).wait()

        for rb in range(_K // _NROWCHAIN):
            rows = [rb * _NROWCHAIN + k for k in range(_NROWCHAIN)]

            init = (tuple((neg_inf, zeros) for _ in range(_NROWCHAIN)),
                    iota)

            @plsc.parallel_loop(0, _STEPS, unroll=_UNROLL, carry=init)
            def step_iter(i, carry, rows=rows, buf=buf):
                accs, cv = carry
                accs = list(accs)
                st = i * _L
                for k in range(_NROWCHAIN):
                    bv, bi = accs[k]
                    v = buf[rows[k], pl.ds(st, _L)]
                    m = v > bv
                    accs[k] = (jnp.where(m, v, bv),
                               jnp.where(m, cv, bi))
                return tuple(accs), cv + _L

            fin, _ = step_iter
            for k in range(_NROWCHAIN):
                bv, bi = fin[k]
                cval_v[rows[k], 0:_L] = bv
                cidx_v[rows[k], 0:_L] = bi

        bv = bi = None
        for c in range(_L):
            cc = jnp.full((_L,), c, jnp.int32)
            v = plsc.load_gather(cval_v, [iota, cc])
            ii = plsc.load_gather(cidx_v, [iota, cc])
            if c == 0:
                bv, bi = v, ii
            else:
                m = (v > bv) | ((v == bv) & (ii < bi))
                bv = jnp.where(m, v, bv)
                bi = jnp.where(m, ii, bi)
        plsc.store_scatter(res_v, [g * _L + iota], bi)

    def outer(t, carry):
        gb = t * _NBUF
        for p in range(_NBUF):
            compute_group(gb + p, p)

            @pl.when(gb + p + _NBUF < _NG)
            def _(p=p):
                dma(gb + p + _NBUF, p).start()

        return carry

    lax.fori_loop(0, _NG // _NBUF, outer, 0)

    pltpu.sync_copy(res_v, out_hbm.at[pl.ds(wid * _BPW, _BPW)])


def _run(obs, tab):
    fn = pl.kernel(
        _sc_body,
        out_type=jax.ShapeDtypeStruct((_BATCH,), jnp.int32),
        mesh=plsc.VectorSubcoreMesh(core_axis_name="c", subcore_axis_name="s"),
        compiler_params=pltpu.CompilerParams(needs_layout_passes=False),
        scratch_types=[
            pltpu.VMEM((_BPW * 2,), jnp.int32),
            pltpu.VMEM((_NG, _L), jnp.int32),
            pltpu.VMEM((_K, _N_ACT), jnp.float32),
            pltpu.VMEM((_K, _N_ACT), jnp.float32),
            pltpu.VMEM((_K, _N_ACT), jnp.float32),
            pltpu.VMEM((_K, _N_ACT), jnp.float32),
            pltpu.VMEM((_K, 17), jnp.float32),
            pltpu.VMEM((_K, 17), jnp.int32),
            pltpu.VMEM((_BPW,), jnp.int32),
            pltpu.SemaphoreType.DMA,
            pltpu.SemaphoreType.DMA,
            pltpu.SemaphoreType.DMA,
            pltpu.SemaphoreType.DMA,
        ],
    )
    return fn(obs, tab)


def kernel(observation, q_table):
    obs = observation.astype(jnp.int32).reshape(_BATCH * 2)
    tab = q_table.reshape(_N_ROWS * _N_COLS, _N_ACT)
    return _run(obs, tab)


# speedup vs baseline: 1.2512x; 1.2512x over previous
import jax
import jax.numpy as jnp
from jax import lax
from jax.experimental import pallas as pl
from jax.experimental.pallas import tpu as pltpu
from jax.experimental.pallas import tpu_sc as plsc

_N_ROWS = 128
_N_COLS = 128
_N_ACT = 1024
_BATCH = 16384

_NC = 2
_NS = 16
_L = 16
_NW = _NC * _NS
_BPW = _BATCH // _NW
_K = 16
_NG = _BPW // _K
_NROWCHAIN = 4
_STEPS = _N_ACT // _L
_UNROLL = 4
_NBUF = 4


def _sc_body(obs_hbm, tab_hbm, out_hbm, obs_v, idx_v, buf0, buf1, buf2, buf3,
             cval_v, cidx_v, res_v, sem0, sem1, sem2, sem3):
    wid = lax.axis_index("s") * _NC + lax.axis_index("c")
    base = wid * _BPW

    pltpu.sync_copy(obs_hbm.at[pl.ds(base * 2, _BPW * 2)], obs_v)

    iota = lax.iota(jnp.int32, _L)

    bufs = (buf0, buf1, buf2, buf3)
    sems = (sem0, sem1, sem2, sem3)

    def dma(g, p):
        return pltpu.make_async_copy(tab_hbm.at[idx_v.at[g]], bufs[p],
                                     sems[p])

    def flat_ids(g):
        rsel = (g * _L + iota) * 2
        r = plsc.load_gather(obs_v, [rsel])
        c = plsc.load_gather(obs_v, [rsel + 1])
        idx_v[g, :] = r * _N_COLS + c

    for g in range(_NBUF):
        flat_ids(g)
    for p in range(_NBUF):
        dma(p, p).start()
    for g in range(_NBUF, _NG):
        flat_ids(g)

    neg_inf = jnp.full((_L,), -jnp.inf, jnp.float32)
    zeros = jnp.zeros((_L,), jnp.int32)

    def compute_group(g, p):
        buf = bufs[p]
        dma(g, p).wait()

        for rb in range(_K // _NROWCHAIN):
            rows = [rb * _NROWCHAIN + k for k in range(_NROWCHAIN)]

            init = (tuple((neg_inf, zeros) for _ in range(_NROWCHAIN)),
                    iota)

            @plsc.parallel_loop(0, _STEPS, unroll=_UNROLL, carry=init)
            def step_iter(i, carry, rows=rows, buf=buf):
                accs, cv = carry
                accs = list(accs)
                st = i * _L
                for k in range(_NROWCHAIN):
                    bv, bi = accs[k]
                    v = buf[rows[k], pl.ds(st, _L)]
                    m = v > bv
                    accs[k] = (jnp.where(m, v, bv),
                               jnp.where(m, cv, bi))
                return tuple(accs), cv + _L

            fin, _ = step_iter
            for k in range(_NROWCHAIN):
                bv, bi = fin[k]
                cval_v[rows[k], 0:_L] = bv
                cidx_v[rows[k], 0:_L] = bi

        bv = bi = None
        for c in range(_L):
            cc = jnp.full((_L,), c, jnp.int32)
            v = plsc.load_gather(cval_v, [iota, cc])
            ii = plsc.load_gather(cidx_v, [iota, cc])
            if c == 0:
                bv, bi = v, ii
            else:
                m = (v > bv) | ((v == bv) & (ii < bi))
                bv = jnp.where(m, v, bv)
                bi = jnp.where(m, ii, bi)
        plsc.store_scatter(res_v, [g * _L + iota], bi)

    def outer(t, carry):
        gb = t * _NBUF
        for p in range(_NBUF):
            compute_group(gb + p, p)

            @pl.when(gb + p + _NBUF < _NG)
            def _(p=p):
                dma(gb + p + _NBUF, p).start()

        return carry

    lax.fori_loop(0, _NG // _NBUF, outer, 0)

    pltpu.sync_copy(res_v, out_hbm.at[pl.ds(wid * _BPW, _BPW)])


def _run(obs, tab):
    fn = pl.kernel(
        _sc_body,
        out_type=jax.ShapeDtypeStruct((_BATCH,), jnp.int32),
        mesh=plsc.VectorSubcoreMesh(core_axis_name="c", subcore_axis_name="s"),
        compiler_params=pltpu.CompilerParams(needs_layout_passes=False),
        scratch_types=[
            pltpu.VMEM((_BPW * 2,), jnp.int32),
            pltpu.VMEM((_NG, _L), jnp.int32),
            pltpu.VMEM((_K, _N_ACT), jnp.float32),
            pltpu.VMEM((_K, _N_ACT), jnp.float32),
            pltpu.VMEM((_K, _N_ACT), jnp.float32),
            pltpu.VMEM((_K, _N_ACT), jnp.float32),
            pltpu.VMEM((_K, 17), jnp.float32),
            pltpu.VMEM((_K, 17), jnp.int32),
            pltpu.VMEM((_BPW,), jnp.int32),
            pltpu.SemaphoreType.DMA,
            pltpu.SemaphoreType.DMA,
            pltpu.SemaphoreType.DMA,
            pltpu.SemaphoreType.DMA,
        ],
    )
    return fn(obs, tab)


def kernel(observation, q_table):
    obs = observation.astype(jnp.int32).reshape(_BATCH * 2)
    tab = q_table.reshape(_N_ROWS * _N_COLS, _N_ACT)
    return _run(obs, tab)


# speedup vs baseline: 1.2514x; 1.0002x over previous
import jax
import jax.numpy as jnp
from jax import lax
from jax.experimental import pallas as pl
from jax.experimental.pallas import tpu as pltpu
from jax.experimental.pallas import tpu_sc as plsc

_N_ROWS = 128
_N_COLS = 128
_N_ACT = 1024
_BATCH = 16384

_NC = 2
_NS = 16
_L = 16
_NW = _NC * _NS
_BPW = _BATCH // _NW
_K = 16
_NG = _BPW // _K
_NROWCHAIN = 4
_STEPS = _N_ACT // _L
_UNROLL = 4
_NBUF = 4


def _sc_body(obs_hbm, tab_hbm, out_hbm, obs_v, idx_v, buf0, buf1, buf2, buf3,
             cval_v, cidx_v, res_v, sem0, sem1, sem2, sem3):
    wid = lax.axis_index("s") * _NC + lax.axis_index("c")
    base = wid * _BPW

    pltpu.sync_copy(obs_hbm.at[pl.ds(base * 2, _BPW * 2)], obs_v)

    iota = lax.iota(jnp.int32, _L)

    bufs = (buf0, buf1, buf2, buf3)
    sems = (sem0, sem1, sem2, sem3)

    def dma(g, p):
        return pltpu.make_async_copy(tab_hbm.at[idx_v.at[g]], bufs[p],
                                     sems[p])

    def flat_ids(g):
        rsel = (g * _L + iota) * 2
        r = plsc.load_gather(obs_v, [rsel])
        c = plsc.load_gather(obs_v, [rsel + 1])
        idx_v[g, :] = r * _N_COLS + c

    for g in range(_NBUF):
        flat_ids(g)
    for p in range(_NBUF):
        dma(p, p).start()
    for g in range(_NBUF, _NG):
        flat_ids(g)

    neg_inf = jnp.full((_L,), -jnp.inf, jnp.float32)
    zeros = jnp.zeros((_L,), jnp.int32)

    def compute_group(g, p):
        buf = bufs[p]
        dma(g, p).wait()

        for rb in range(_K // _NROWCHAIN):
            rows = [rb * _NROWCHAIN + k for k in range(_NROWCHAIN)]

            init = (tuple((neg_inf, zeros) for _ in range(_NROWCHAIN)),
                    iota)

            @plsc.parallel_loop(0, _STEPS, unroll=_UNROLL, carry=init)
            def step_iter(i, carry, rows=rows, buf=buf):
                accs, cv = carry
                accs = list(accs)
                st = i * _L
                for k in range(_NROWCHAIN):
                    bv, bi = accs[k]
                    v = buf[rows[k], pl.ds(st, _L)]
                    m = v > bv
                    accs[k] = (jnp.where(m, v, bv),
                               jnp.where(m, cv, bi))
                return tuple(accs), cv + _L

            fin, _ = step_iter
            for k in range(_NROWCHAIN):
                bv, bi = fin[k]
                cval_v[rows[k], 0:_L] = bv
                cidx_v[rows[k], 0:_L] = bi

        cands = []
        for c in range(_L):
            cc = jnp.full((_L,), c, jnp.int32)
            cands.append((plsc.load_gather(cval_v, [iota, cc]),
                          plsc.load_gather(cidx_v, [iota, cc])))
        while len(cands) > 1:
            nxt = []
            for a in range(0, len(cands), 2):
                (va, ia), (vb, ib) = cands[a], cands[a + 1]
                m = (vb > va) | ((vb == va) & (ib < ia))
                nxt.append((jnp.where(m, vb, va), jnp.where(m, ib, ia)))
            cands = nxt
        plsc.store_scatter(res_v, [g * _L + iota], cands[0][1])

    def outer(t, carry):
        gb = t * _NBUF
        for p in range(_NBUF):
            compute_group(gb + p, p)

            @pl.when(gb + p + _NBUF < _NG)
            def _(p=p):
                dma(gb + p + _NBUF, p).start()

        return carry

    lax.fori_loop(0, _NG // _NBUF, outer, 0)

    pltpu.sync_copy(res_v, out_hbm.at[pl.ds(wid * _BPW, _BPW)])


def _run(obs, tab):
    fn = pl.kernel(
        _sc_body,
        out_type=jax.ShapeDtypeStruct((_BATCH,), jnp.int32),
        mesh=plsc.VectorSubcoreMesh(core_axis_name="c", subcore_axis_name="s"),
        compiler_params=pltpu.CompilerParams(needs_layout_passes=False),
        scratch_types=[
            pltpu.VMEM((_BPW * 2,), jnp.int32),
            pltpu.VMEM((_NG, _L), jnp.int32),
            pltpu.VMEM((_K, _N_ACT), jnp.float32),
            pltpu.VMEM((_K, _N_ACT), jnp.float32),
            pltpu.VMEM((_K, _N_ACT), jnp.float32),
            pltpu.VMEM((_K, _N_ACT), jnp.float32),
            pltpu.VMEM((_K, 17), jnp.float32),
            pltpu.VMEM((_K, 17), jnp.int32),
            pltpu.VMEM((_BPW,), jnp.int32),
            pltpu.SemaphoreType.DMA,
            pltpu.SemaphoreType.DMA,
            pltpu.SemaphoreType.DMA,
            pltpu.SemaphoreType.DMA,
        ],
    )
    return fn(obs, tab)


def kernel(observation, q_table):
    obs = observation.astype(jnp.int32).reshape(_BATCH * 2)
    tab = q_table.reshape(_N_ROWS * _N_COLS, _N_ACT)
    return _run(obs, tab)


# speedup vs baseline: 1.2892x; 1.0302x over previous
import jax
import jax.numpy as jnp
from jax import lax
from jax.experimental import pallas as pl
from jax.experimental.pallas import tpu as pltpu
from jax.experimental.pallas import tpu_sc as plsc

_N_ROWS = 128
_N_COLS = 128
_N_ACT = 1024
_BATCH = 16384

_NC = 2
_NS = 16
_L = 16
_NW = _NC * _NS
_BPW = _BATCH // _NW
_K = 16
_NG = _BPW // _K
_NROWCHAIN = 8
_STEPS = _N_ACT // _L
_UNROLL = 2
_NBUF = 4


def _sc_body(obs_hbm, tab_hbm, out_hbm, obs_v, idx_v, buf0, buf1, buf2, buf3,
             cval_v, cidx_v, res_v, sem0, sem1, sem2, sem3):
    wid = lax.axis_index("s") * _NC + lax.axis_index("c")
    base = wid * _BPW

    pltpu.sync_copy(obs_hbm.at[pl.ds(base * 2, _BPW * 2)], obs_v)

    iota = lax.iota(jnp.int32, _L)

    bufs = (buf0, buf1, buf2, buf3)
    sems = (sem0, sem1, sem2, sem3)

    def dma(g, p):
        return pltpu.make_async_copy(tab_hbm.at[idx_v.at[g]], bufs[p],
                                     sems[p])

    def flat_ids(g):
        rsel = (g * _L + iota) * 2
        r = plsc.load_gather(obs_v, [rsel])
        c = plsc.load_gather(obs_v, [rsel + 1])
        idx_v[g, :] = r * _N_COLS + c

    for g in range(_NBUF):
        flat_ids(g)
    for p in range(_NBUF):
        dma(p, p).start()
    for g in range(_NBUF, _NG):
        flat_ids(g)

    neg_inf = jnp.full((_L,), -jnp.inf, jnp.float32)
    zeros = jnp.zeros((_L,), jnp.int32)

    def compute_group(g, p):
        buf = bufs[p]
        dma(g, p).wait()

        for rb in range(_K // _NROWCHAIN):
            rows = [rb * _NROWCHAIN + k for k in range(_NROWCHAIN)]

            init = (tuple((neg_inf, zeros) for _ in range(_NROWCHAIN)),
                    iota)

            @plsc.parallel_loop(0, _STEPS, unroll=_UNROLL, carry=init)
            def step_iter(i, carry, rows=rows, buf=buf):
                accs, cv = carry
                accs = list(accs)
                st = i * _L
                for k in range(_NROWCHAIN):
                    bv, bi = accs[k]
                    v = buf[rows[k], pl.ds(st, _L)]
                    m = v > bv
                    accs[k] = (jnp.where(m, v, bv),
                               jnp.where(m, cv, bi))
                return tuple(accs), cv + _L

            fin, _ = step_iter
            for k in range(_NROWCHAIN):
                bv, bi = fin[k]
                cval_v[rows[k], 0:_L] = bv
                cidx_v[rows[k], 0:_L] = bi

        cands = []
        for c in range(_L):
            cc = jnp.full((_L,), c, jnp.int32)
            cands.append((plsc.load_gather(cval_v, [iota, cc]),
                          plsc.load_gather(cidx_v, [iota, cc])))
        while len(cands) > 1:
            nxt = []
            for a in range(0, len(cands), 2):
                (va, ia), (vb, ib) = cands[a], cands[a + 1]
                m = (vb > va) | ((vb == va) & (ib < ia))
                nxt.append((jnp.where(m, vb, va), jnp.where(m, ib, ia)))
            cands = nxt
        plsc.store_scatter(res_v, [g * _L + iota], cands[0][1])

    def outer(t, carry):
        gb = t * _NBUF
        for p in range(_NBUF):
            compute_group(gb + p, p)

            @pl.when(gb + p + _NBUF < _NG)
            def _(p=p):
                dma(gb + p + _NBUF, p).start()

        return carry

    lax.fori_loop(0, _NG // _NBUF, outer, 0)

    pltpu.sync_copy(res_v, out_hbm.at[pl.ds(wid * _BPW, _BPW)])


def _run(obs, tab):
    fn = pl.kernel(
        _sc_body,
        out_type=jax.ShapeDtypeStruct((_BATCH,), jnp.int32),
        mesh=plsc.VectorSubcoreMesh(core_axis_name="c", subcore_axis_name="s"),
        compiler_params=pltpu.CompilerParams(needs_layout_passes=False),
        scratch_types=[
            pltpu.VMEM((_BPW * 2,), jnp.int32),
            pltpu.VMEM((_NG, _L), jnp.int32),
            pltpu.VMEM((_K, _N_ACT), jnp.float32),
            pltpu.VMEM((_K, _N_ACT), jnp.float32),
            pltpu.VMEM((_K, _N_ACT), jnp.float32),
            pltpu.VMEM((_K, _N_ACT), jnp.float32),
            pltpu.VMEM((_K, 17), jnp.float32),
            pltpu.VMEM((_K, 17), jnp.int32),
            pltpu.VMEM((_BPW,), jnp.int32),
            pltpu.SemaphoreType.DMA,
            pltpu.SemaphoreType.DMA,
            pltpu.SemaphoreType.DMA,
            pltpu.SemaphoreType.DMA,
        ],
    )
    return fn(obs, tab)


def kernel(observation, q_table):
    obs = observation.astype(jnp.int32).reshape(_BATCH * 2)
    tab = q_table.reshape(_N_ROWS * _N_COLS, _N_ACT)
    return _run(obs, tab)
